# baseline (speedup 1.0000x reference)
import functools

import jax
import jax.numpy as jnp
from jax import lax
from jax.experimental import pallas as pl
from jax.experimental.pallas import tpu as pltpu
from jax.experimental.pallas import tpu_sc as plsc

N_USER = 10000
N_REST = 10000
N_CAT = 512
D = 128
D_EDGE = 16
E_UR = 320000
E_RC = 40000

NTILES = 32
LANES = 16
B = 80

N_PAD_RU = 10112
N_PAD_C = 640
E_RC_PAD = 40960


_SC_MESH = plsc.VectorSubcoreMesh(core_axis_name="c", subcore_axis_name="s",
                                  num_cores=2, num_subcores=16)
_SC_PARAMS = pltpu.CompilerParams(use_tc_tiling_on_sc=False,
                                  needs_layout_passes=False)


def _zero_stripe(rows0_v, ex0_v, acc_s, den_s, sid, rz):
    def zrow(r, c):
        for k in range(D // LANES):
            rows0_v[r, pl.ds(k * LANES, LANES)] = jnp.zeros(
                (LANES,), jnp.float32)
        return c
    lax.fori_loop(0, B, zrow, 0)
    for k in range(B // LANES):
        ex0_v[pl.ds(k * LANES, LANES)] = jnp.zeros((LANES,), jnp.float32)
    base = sid * rz
    off = 0
    while off < rz:
        n = min(B, rz - off)
        pltpu.sync_copy(rows0_v.at[pl.ds(0, n)],
                        acc_s.at[pl.ds(base + off, n)])
        pltpu.sync_copy(ex0_v.at[pl.ds(0, n)],
                        den_s.at[pl.ds(base + off, n)])
        off += n


def _dump_stripe(acc_s, den_s, acc_out, den_out, cid, sid, rz):
    pltpu.sync_copy(acc_s.at[pl.ds(sid * rz, rz)],
                    acc_out.at[cid, pl.ds(sid * rz, rz)])
    pltpu.sync_copy(den_s.at[pl.ds(sid * rz, rz)],
                    den_out.at[cid, pl.ds(sid * rz, rz)])


def _make_gat_pair(n_pad, e_total):
    e_t = e_total // NTILES
    steps = e_t // B
    ch = 10
    nch = steps // ch
    tail = steps - nch * ch
    rz = n_pad // 16

    @functools.partial(
        pl.kernel,
        out_type=[jax.ShapeDtypeStruct((2, n_pad, D), jnp.float32),
                  jax.ShapeDtypeStruct((2, n_pad), jnp.float32),
                  jax.ShapeDtypeStruct((2, 16), jnp.float32)] * 2,
        mesh=_SC_MESH,
        scratch_types=[
            pltpu.VMEM((N_REST,), jnp.float32),
            pltpu.VMEM((N_REST,), jnp.float32),
            pltpu.VMEM((ch, 3, B), jnp.int32),
            pltpu.VMEM((B,), jnp.float32),
            pltpu.VMEM((B,), jnp.float32),
            pltpu.VMEM((B, D), jnp.float32),
            pltpu.VMEM((B, D), jnp.float32),
            pltpu.VMEM((16,), jnp.float32),
            pltpu.VMEM((16, 16), jnp.float32),
            pltpu.VMEM_SHARED((16, 16), jnp.float32),
            pltpu.VMEM_SHARED((n_pad, D), jnp.float32),
            pltpu.VMEM_SHARED((n_pad,), jnp.float32),
            pltpu.SemaphoreType.DMA,
            pltpu.SemaphoreType.DMA,
            pltpu.SemaphoreType.DMA,
            pltpu.SemaphoreType.DMA,
        ],
        compiler_params=_SC_PARAMS,
    )
    def gat_pair(ed_a, asrc_a, adst_a, h_a, ed_b, asrc_b, adst_b, h_b,
                 acc_oa, den_oa, g_oa, acc_ob, den_ob, g_ob,
                 asrc_v, adst_v, edc_v, ex0_v, ex1_v, rows0_v, rows1_v,
                 tmp16_v, gmax_l, gmax_s, acc_s, den_s,
                 semg0, semg1, sems0, sems1):
        cid = lax.axis_index("c")
        sid = lax.axis_index("s")
        wid = cid * 16 + sid
        exv = (ex0_v, ex1_v)
        rowsv = (rows0_v, rows1_v)
        semg = (semg0, semg1)
        sems = (sems0, sems1)

        def section(ed_hbm, asrc_hbm, adst_hbm, h_hbm,
                    acc_out, den_out, g_out):
            pltpu.sync_copy(asrc_hbm, asrc_v)
            pltpu.sync_copy(adst_hbm, adst_v)
            _zero_stripe(rows0_v, ex0_v, acc_s, den_s, sid, rz)

            def alpha16(j, k):
                i16 = edc_v[j, 0, pl.ds(k * LANES, LANES)]
                d16 = edc_v[j, 1, pl.ds(k * LANES, LANES)]
                ae = plsc.bitcast(edc_v[j, 2, pl.ds(k * LANES, LANES)],
                                  jnp.float32)
                a = (plsc.load_gather(asrc_v, [i16])
                     + plsc.load_gather(adst_v, [d16]) + ae)
                return jnp.maximum(a, 0.0) + 0.2 * jnp.minimum(a, 0.0)

            def p1_chunk(cbase, mx, nsteps):
                pltpu.sync_copy(ed_hbm.at[wid, pl.ds(cbase, nsteps)],
                                edc_v.at[pl.ds(0, nsteps)])
                for j in range(nsteps):
                    for k in range(B // LANES):
                        mx = jnp.maximum(mx, alpha16(j, k))
                return mx
            mx = lax.fori_loop(
                0, nch, lambda cc, mx: p1_chunk(cc * ch, mx, ch),
                jnp.full((LANES,), -1e30, dtype=jnp.float32))
            if tail:
                mx = p1_chunk(nch * ch, mx, tail)
            tmp16_v[...] = mx
            pltpu.sync_copy(tmp16_v, gmax_s.at[sid])
            plsc.subcore_barrier()
            pltpu.sync_copy(gmax_s, gmax_l)
            gm = gmax_l[0, pl.ds(0, LANES)]
            for r in range(1, 16):
                gm = jnp.maximum(gm, gmax_l[r, pl.ds(0, LANES)])
            G = lax.reduce_max(gm, axes=(0,))

            @pl.when(sid == 0)
            def _():
                tmp16_v[...] = jnp.full((LANES,), G, dtype=jnp.float32)
                pltpu.sync_copy(tmp16_v, g_out.at[cid])

            def p2_chunk(cbase, nsteps):
                pltpu.sync_copy(ed_hbm.at[wid, pl.ds(cbase, nsteps)],
                                edc_v.at[pl.ds(0, nsteps)])
                gdesc = [None, None]
                sdesc = [None, None]
                gdesc[0] = pltpu.async_copy(h_hbm.at[edc_v.at[0, 0]],
                                            rows0_v, semg[0])
                for j in range(nsteps):
                    buf = j % 2
                    if j + 1 < nsteps:
                        if sdesc[1 - buf] is not None:
                            for dd in sdesc[1 - buf]:
                                dd.wait()
                            sdesc[1 - buf] = None
                        gdesc[1 - buf] = pltpu.async_copy(
                            h_hbm.at[edc_v.at[j + 1, 0]], rowsv[1 - buf],
                            semg[1 - buf])
                    gdesc[buf].wait()
                    for k in range(B // LANES):
                        exv[buf][pl.ds(k * LANES, LANES)] = jnp.exp(
                            alpha16(j, k) - G)

                    def prow(r2, cc2, _buf=buf):
                        for u in range(2):
                            exb = plsc.load_gather(
                                exv[_buf],
                                [jnp.full((LANES,), 2 * r2 + u,
                                          dtype=jnp.int32)])
                            for k in range(D // LANES):
                                rowsv[_buf][
                                    2 * r2 + u, pl.ds(k * LANES, LANES)] = (
                                    rowsv[_buf][2 * r2 + u,
                                                pl.ds(k * LANES, LANES)]
                                    * exb)
                        return cc2
                    lax.fori_loop(0, B // 2, prow, 0)
                    sdesc[buf] = (
                        pltpu.async_copy(rowsv[buf],
                                         acc_s.at[edc_v.at[j, 1]],
                                         sems[buf], add=True),
                        pltpu.async_copy(exv[buf],
                                         den_s.at[edc_v.at[j, 1]],
                                         sems[buf], add=True))
                for b in (0, 1):
                    if sdesc[b] is not None:
                        for dd in sdesc[b]:
                            dd.wait()
            lax.fori_loop(0, nch,
                          lambda cc, c: (p2_chunk(cc * ch, ch), c)[1], 0)
            if tail:
                p2_chunk(nch * ch, tail)

            plsc.subcore_barrier()
            _dump_stripe(acc_s, den_s, acc_out, den_out, cid, sid, rz)

        section(ed_a, asrc_a, adst_a, h_a, acc_oa, den_oa, g_oa)
        plsc.subcore_barrier()
        section(ed_b, asrc_b, adst_b, h_b, acc_ob, den_ob, g_ob)

    return gat_pair


def _make_sage_pair():
    steps = E_RC_PAD // NTILES // B
    rz_a = N_PAD_RU // 16
    rz_b = N_PAD_C // 16

    @functools.partial(
        pl.kernel,
        out_type=[jax.ShapeDtypeStruct((2, N_PAD_RU, D), jnp.float32),
                  jax.ShapeDtypeStruct((2, N_PAD_RU), jnp.float32),
                  jax.ShapeDtypeStruct((2, N_PAD_C, D), jnp.float32),
                  jax.ShapeDtypeStruct((2, N_PAD_C), jnp.float32)],
        mesh=_SC_MESH,
        scratch_types=[
            pltpu.VMEM((steps, 3, B), jnp.int32),
            pltpu.VMEM((B,), jnp.float32),
            pltpu.VMEM((B,), jnp.float32),
            pltpu.VMEM((B, D), jnp.float32),
            pltpu.VMEM((B, D), jnp.float32),
            pltpu.VMEM_SHARED((N_PAD_RU, D), jnp.float32),
            pltpu.VMEM_SHARED((N_PAD_RU,), jnp.float32),
            pltpu.SemaphoreType.DMA,
            pltpu.SemaphoreType.DMA,
            pltpu.SemaphoreType.DMA,
            pltpu.SemaphoreType.DMA,
        ],
        compiler_params=_SC_PARAMS,
    )
    def sage_pair(ed_a, x_a, ed_b, x_b, acc_oa, den_oa, acc_ob, den_ob,
                  edc_v, ones_v, zeros_v, rows0_v, rows1_v, acc_s, den_s,
                  semg0, semg1, sems0, sems1):
        cid = lax.axis_index("c")
        sid = lax.axis_index("s")
        wid = cid * 16 + sid
        rowsv = (rows0_v, rows1_v)
        semg = (semg0, semg1)
        sems = (sems0, sems1)
        for k in range(B // LANES):
            ones_v[pl.ds(k * LANES, LANES)] = jnp.full((LANES,), 1.0,
                                                       jnp.float32)

        def section(ed_hbm, x_hbm, acc_out, den_out, rz):
            _zero_stripe(rows0_v, zeros_v, acc_s, den_s, sid, rz)
            plsc.subcore_barrier()
            pltpu.sync_copy(ed_hbm.at[wid], edc_v)
            gdesc = [None, None]
            sdesc = [None, None]
            gdesc[0] = pltpu.async_copy(x_hbm.at[edc_v.at[0, 0]],
                                        rows0_v, semg[0])
            for j in range(steps):
                buf = j % 2
                if j + 1 < steps:
                    if sdesc[1 - buf] is not None:
                        for dd in sdesc[1 - buf]:
                            dd.wait()
                        sdesc[1 - buf] = None
                    gdesc[1 - buf] = pltpu.async_copy(
                        x_hbm.at[edc_v.at[j + 1, 0]], rowsv[1 - buf],
                        semg[1 - buf])
                gdesc[buf].wait()
                sdesc[buf] = (
                    pltpu.async_copy(rowsv[buf], acc_s.at[edc_v.at[j, 1]],
                                     sems[buf], add=True),
                    pltpu.async_copy(ones_v, den_s.at[edc_v.at[j, 1]],
                                     sems[buf], add=True))
            for b in (0, 1):
                if sdesc[b] is not None:
                    for dd in sdesc[b]:
                        dd.wait()
            plsc.subcore_barrier()
            _dump_stripe(acc_s, den_s, acc_out, den_out, cid, sid, rz)

        section(ed_a, x_a, acc_oa, den_oa, rz_a)
        plsc.subcore_barrier()
        section(ed_b, x_b, acc_ob, den_ob, rz_b)

    return sage_pair


_gat_pair_sc = _make_gat_pair(N_PAD_RU, E_UR)
_sage_pair_sc = _make_sage_pair()


def _prep_body(xs_ref, xd_ref, ws_ref, avs_ref, wd_ref, avd_ref,
               h_ref, asrc_ref, adst_ref):
    h = jnp.dot(xs_ref[...], ws_ref[...], preferred_element_type=jnp.float32)
    h_ref[...] = h
    asrc_ref[...] = jnp.dot(h, avs_ref[...], preferred_element_type=jnp.float32)
    wv = jnp.dot(wd_ref[...], avd_ref[...], preferred_element_type=jnp.float32)
    adst_ref[...] = jnp.dot(xd_ref[...], wv, preferred_element_type=jnp.float32)


def _tc_prep(x_src, x_dst, p):
    n_s, n_d = x_src.shape[0], x_dst.shape[0]
    h, a_s, a_d = pl.pallas_call(
        _prep_body,
        out_shape=[jax.ShapeDtypeStruct((n_s, D), jnp.float32),
                   jax.ShapeDtypeStruct((n_s, 1), jnp.float32),
                   jax.ShapeDtypeStruct((n_d, 1), jnp.float32)],
    )(x_src, x_dst, p['W_src'], p['att_src'].reshape(D, 1),
      p['W_dst'], p['att_dst'].reshape(D, 1))
    return h, a_s.reshape(n_s), a_d.reshape(n_d)


def _edge_body(ea_ref, we_ref, ave_ref, out_ref):
    we = jnp.dot(we_ref[...], ave_ref[...],
                 preferred_element_type=jnp.float32)
    wet = jnp.concatenate([we] * 8, axis=0)
    r = lax.broadcasted_iota(jnp.int32, (128, 8), 0)
    cidx = lax.broadcasted_iota(jnp.int32, (128, 8), 1)
    wsel = jnp.where(r // 16 == cidx, wet, 0.0)
    out_ref[...] = jnp.dot(ea_ref[...], wsel,
                           preferred_element_type=jnp.float32)


def _tc_edge(edge_attr, p):
    e = edge_attr.shape[0]
    ea2 = edge_attr.reshape(e // 8, 128)
    out = pl.pallas_call(
        _edge_body,
        out_shape=jax.ShapeDtypeStruct((e // 8, 8), jnp.float32),
    )(ea2, p['W_edge'], p['att_edge'].reshape(D, 1))
    return out.reshape(e)


def _gat_from_acc(gacc_ref, gden_ref, g_ref, n):
    g0 = g_ref[0, 0]
    g1 = g_ref[1, 0]
    gm = jnp.maximum(g0, g1)
    s0 = jnp.exp(g0 - gm)
    s1 = jnp.exp(g1 - gm)
    f = gacc_ref[0, :n, :] * s0 + gacc_ref[1, :n, :] * s1
    den = gden_ref[0, :n, :] * s0 + gden_ref[1, :n, :] * s1
    return f / (den + 1e-16)


def _sage_from_acc(sacc_ref, scnt_ref, x_ref, wl_ref, wr_ref, bs_ref, n):
    ss = sacc_ref[0, :n, :] + sacc_ref[1, :n, :]
    cnt = scnt_ref[0, :n, :] + scnt_ref[1, :n, :]
    mean = ss / jnp.maximum(cnt, 1.0)
    return (jnp.dot(mean, wl_ref[...], preferred_element_type=jnp.float32)
            + jnp.dot(x_ref[...], wr_ref[...],
                      preferred_element_type=jnp.float32)
            + bs_ref[...])


def _bn_relu(y, gamma_ref, beta_ref):
    mu = jnp.mean(y, axis=0, keepdims=True)
    var = jnp.mean((y - mu) * (y - mu), axis=0, keepdims=True)
    out = (y - mu) / jnp.sqrt(var + 1e-5) * gamma_ref[...] + beta_ref[...]
    return jnp.maximum(out, 0.0)


def _comb_rest_body(gacc_ref, gden_ref, g_ref, sacc_ref, scnt_ref, x_ref,
                    bg_ref, wl_ref, wr_ref, bs_ref, gamma_ref, beta_ref,
                    out_ref):
    gat = _gat_from_acc(gacc_ref, gden_ref, g_ref, N_REST) + bg_ref[...]
    sage = _sage_from_acc(sacc_ref, scnt_ref, x_ref, wl_ref, wr_ref, bs_ref,
                          N_REST)
    out_ref[...] = _bn_relu(gat + sage, gamma_ref, beta_ref)


def _comb_user_body(gacc_ref, gden_ref, g_ref, bg_ref, gamma_ref, beta_ref,
                    out_ref):
    gat = _gat_from_acc(gacc_ref, gden_ref, g_ref, N_USER) + bg_ref[...]
    out_ref[...] = _bn_relu(gat, gamma_ref, beta_ref)


def _comb_cat_body(sacc_ref, scnt_ref, x_ref, wl_ref, wr_ref, bs_ref,
                   gamma_ref, beta_ref, out_ref):
    sage = _sage_from_acc(sacc_ref, scnt_ref, x_ref, wl_ref, wr_ref, bs_ref,
                          N_CAT)
    out_ref[...] = _bn_relu(sage, gamma_ref, beta_ref)


_SMEM_SPEC = pl.BlockSpec(memory_space=pltpu.SMEM)


def _col(den):
    return den.reshape(den.shape[0], den.shape[1], 1)


def _comb_rest(gacc, gden, g2, sacc, scnt, x, pg, ps, bn):
    return pl.pallas_call(
        _comb_rest_body,
        out_shape=jax.ShapeDtypeStruct((N_REST, D), jnp.float32),
        in_specs=[pl.BlockSpec(None), pl.BlockSpec(None), _SMEM_SPEC]
        + [pl.BlockSpec(None)] * 9,
    )(gacc, _col(gden), g2, sacc, _col(scnt), x, pg['bias'].reshape(1, D),
      ps['W_l'], ps['W_r'], ps['bias'].reshape(1, D),
      bn['gamma'].reshape(1, D), bn['beta'].reshape(1, D))


def _comb_user(gacc, gden, g2, pg, bn):
    return pl.pallas_call(
        _comb_user_body,
        out_shape=jax.ShapeDtypeStruct((N_USER, D), jnp.float32),
        in_specs=[pl.BlockSpec(None), pl.BlockSpec(None), _SMEM_SPEC]
        + [pl.BlockSpec(None)] * 3,
    )(gacc, _col(gden), g2, pg['bias'].reshape(1, D),
      bn['gamma'].reshape(1, D), bn['beta'].reshape(1, D))


def _comb_cat(sacc, scnt, x, ps, bn):
    return pl.pallas_call(
        _comb_cat_body,
        out_shape=jax.ShapeDtypeStruct((N_CAT, D), jnp.float32),
    )(sacc, _col(scnt), x, ps['W_l'], ps['W_r'], ps['bias'].reshape(1, D),
      bn['gamma'].reshape(1, D), bn['beta'].reshape(1, D))


def _pack_edges(src, dst, ae):
    steps = src.shape[0] // (NTILES * B)
    return jnp.stack(
        [src.astype(jnp.int32).reshape(NTILES, steps, B),
         dst.astype(jnp.int32).reshape(NTILES, steps, B),
         lax.bitcast_convert_type(ae, jnp.int32).reshape(NTILES, steps, B)],
        axis=2)


def _pad_idx(idx, fill):
    pad = E_RC_PAD - E_RC
    return jnp.concatenate([idx.astype(jnp.int32),
                            jnp.full((pad,), fill, dtype=jnp.int32)])


def kernel(x_user, x_restaurant, x_category, edge_index_ur, edge_index_ru,
           edge_index_rc, edge_index_cr, edge_attr_ur, edge_attr_ru, params):
    src_cr = _pad_idx(edge_index_cr[0], 0)
    dst_cr = _pad_idx(edge_index_cr[1], N_REST)
    src_rc = _pad_idx(edge_index_rc[0], 0)
    dst_rc = _pad_idx(edge_index_rc[1], N_CAT)
    z_ae = jnp.zeros((E_RC_PAD,), jnp.float32)
    ed_cr = _pack_edges(src_cr, dst_cr, z_ae)
    ed_rc = _pack_edges(src_rc, dst_rc, z_ae)

    xu, xr, xc = x_user, x_restaurant, x_category
    for lname in ('l1', 'l2'):
        P = params[lname]
        h_u, as_u, ad_r = _tc_prep(xu, xr, P['gat_ur'])
        h_r, as_r, ad_u = _tc_prep(xr, xu, P['gat_ru'])
        ae_ur = _tc_edge(edge_attr_ur, P['gat_ur'])
        ae_ru = _tc_edge(edge_attr_ru, P['gat_ru'])
        ed_ur = _pack_edges(edge_index_ur[0], edge_index_ur[1], ae_ur)
        ed_ru = _pack_edges(edge_index_ru[0], edge_index_ru[1], ae_ru)

        (gacc_r, gden_r, g_r, gacc_u, gden_u, g_u) = _gat_pair_sc(
            ed_ur, as_u, ad_r, h_u, ed_ru, as_r, ad_u, h_r)
        sacc_r, scnt_r, sacc_c, scnt_c = _sage_pair_sc(ed_cr, xc, ed_rc, xr)

        xr_new = _comb_rest(gacc_r, gden_r, g_r, sacc_r, scnt_r, xr,
                            P['gat_ur'], P['sage_cr'], P['bn']['restaurant'])
        xu_new = _comb_user(gacc_u, gden_u, g_u, P['gat_ru'], P['bn']['user'])
        xc_new = _comb_cat(sacc_c, scnt_c, xc, P['sage_rc'],
                           P['bn']['category'])
        xu, xr, xc = xu_new, xr_new, xc_new

    return (xu, xr, xc)


# speedup vs baseline: 1.1337x; 1.1337x over previous
import functools

import jax
import jax.numpy as jnp
from jax import lax
from jax.experimental import pallas as pl
from jax.experimental.pallas import tpu as pltpu
from jax.experimental.pallas import tpu_sc as plsc

N_USER = 10000
N_REST = 10000
N_CAT = 512
D = 128
D_EDGE = 16
E_UR = 320000
E_RC = 40000

NTILES = 32
LANES = 16
B = 80

N_PAD_RU = 10112
N_PAD_C = 640
E_RC_PAD = 40960


_SC_MESH = plsc.VectorSubcoreMesh(core_axis_name="c", subcore_axis_name="s",
                                  num_cores=2, num_subcores=16)
_SC_PARAMS = pltpu.CompilerParams(use_tc_tiling_on_sc=False,
                                  needs_layout_passes=False)


def _zero_stripe(rows0_v, ex0_v, acc_s, den_s, sid, rz):
    def zrow(r, c):
        for k in range(D // LANES):
            rows0_v[r, pl.ds(k * LANES, LANES)] = jnp.zeros(
                (LANES,), jnp.float32)
        return c
    lax.fori_loop(0, B, zrow, 0)
    for k in range(B // LANES):
        ex0_v[pl.ds(k * LANES, LANES)] = jnp.zeros((LANES,), jnp.float32)
    base = sid * rz
    off = 0
    while off < rz:
        n = min(B, rz - off)
        pltpu.sync_copy(rows0_v.at[pl.ds(0, n)],
                        acc_s.at[pl.ds(base + off, n)])
        pltpu.sync_copy(ex0_v.at[pl.ds(0, n)],
                        den_s.at[pl.ds(base + off, n)])
        off += n


def _dump_stripe(acc_s, den_s, acc_out, den_out, cid, sid, rz):
    pltpu.sync_copy(acc_s.at[pl.ds(sid * rz, rz)],
                    acc_out.at[cid, pl.ds(sid * rz, rz)])
    pltpu.sync_copy(den_s.at[pl.ds(sid * rz, rz)],
                    den_out.at[cid, pl.ds(sid * rz, rz)])


def _make_gat_sc(n_src, n_dst, n_pad, e_total):
    e_t = e_total // NTILES
    steps = e_t // B
    ch = 10 if steps >= 10 else 8
    nch = steps // ch
    tail = steps - nch * ch
    rz = n_pad // 16

    @functools.partial(
        pl.kernel,
        out_type=[jax.ShapeDtypeStruct((2, n_pad, D), jnp.float32),
                  jax.ShapeDtypeStruct((2, n_pad), jnp.float32),
                  jax.ShapeDtypeStruct((2, 16), jnp.float32)],
        mesh=_SC_MESH,
        scratch_types=[
            pltpu.VMEM((n_src,), jnp.float32),
            pltpu.VMEM((n_dst,), jnp.float32),
            pltpu.VMEM((ch, 3, B), jnp.int32),
            pltpu.VMEM((B,), jnp.float32),
            pltpu.VMEM((B,), jnp.float32),
            pltpu.VMEM((B, D), jnp.float32),
            pltpu.VMEM((B, D), jnp.float32),
            pltpu.VMEM((16,), jnp.float32),
            pltpu.VMEM((16, 16), jnp.float32),
            pltpu.VMEM_SHARED((16, 16), jnp.float32),
            pltpu.VMEM_SHARED((n_pad, D), jnp.float32),
            pltpu.VMEM_SHARED((n_pad,), jnp.float32),
            pltpu.SemaphoreType.DMA,
            pltpu.SemaphoreType.DMA,
            pltpu.SemaphoreType.DMA,
            pltpu.SemaphoreType.DMA,
        ],
        compiler_params=_SC_PARAMS,
    )
    def gat_sc(ed_hbm, asrc_hbm, adst_hbm, h_hbm,
               acc_out, den_out, g_out,
               asrc_v, adst_v, edc_v, ex0_v, ex1_v, rows0_v, rows1_v,
               tmp16_v, gmax_l, gmax_s, acc_s, den_s,
               semg0, semg1, sems0, sems1):
        cid = lax.axis_index("c")
        sid = lax.axis_index("s")
        wid = cid * 16 + sid
        exv = (ex0_v, ex1_v)
        rowsv = (rows0_v, rows1_v)
        semg = (semg0, semg1)
        sems = (sems0, sems1)

        if True:
            pltpu.sync_copy(asrc_hbm, asrc_v)
            pltpu.sync_copy(adst_hbm, adst_v)
            _zero_stripe(rows0_v, ex0_v, acc_s, den_s, sid, rz)

            def alpha16(j, k):
                i16 = edc_v[j, 0, pl.ds(k * LANES, LANES)]
                d16 = edc_v[j, 1, pl.ds(k * LANES, LANES)]
                ae = plsc.bitcast(edc_v[j, 2, pl.ds(k * LANES, LANES)],
                                  jnp.float32)
                a = (plsc.load_gather(asrc_v, [i16])
                     + plsc.load_gather(adst_v, [d16]) + ae)
                return jnp.maximum(a, 0.0) + 0.2 * jnp.minimum(a, 0.0)

            def p1_chunk(cbase, mx, nsteps):
                pltpu.sync_copy(ed_hbm.at[wid, pl.ds(cbase, nsteps)],
                                edc_v.at[pl.ds(0, nsteps)])
                for j in range(nsteps):
                    for k in range(B // LANES):
                        mx = jnp.maximum(mx, alpha16(j, k))
                return mx
            mx = lax.fori_loop(
                0, nch, lambda cc, mx: p1_chunk(cc * ch, mx, ch),
                jnp.full((LANES,), -1e30, dtype=jnp.float32))
            if tail:
                mx = p1_chunk(nch * ch, mx, tail)
            tmp16_v[...] = mx
            pltpu.sync_copy(tmp16_v, gmax_s.at[sid])
            plsc.subcore_barrier()
            pltpu.sync_copy(gmax_s, gmax_l)
            gm = gmax_l[0, pl.ds(0, LANES)]
            for r in range(1, 16):
                gm = jnp.maximum(gm, gmax_l[r, pl.ds(0, LANES)])
            G = lax.reduce_max(gm, axes=(0,))

            @pl.when(sid == 0)
            def _():
                tmp16_v[...] = jnp.full((LANES,), G, dtype=jnp.float32)
                pltpu.sync_copy(tmp16_v, g_out.at[cid])

            def p2_chunk(cbase, nsteps):
                pltpu.sync_copy(ed_hbm.at[wid, pl.ds(cbase, nsteps)],
                                edc_v.at[pl.ds(0, nsteps)])
                gdesc = [None, None]
                sdesc = [None, None]
                gdesc[0] = pltpu.async_copy(h_hbm.at[edc_v.at[0, 0]],
                                            rows0_v, semg[0])
                for j in range(nsteps):
                    buf = j % 2
                    if j + 1 < nsteps:
                        if sdesc[1 - buf] is not None:
                            for dd in sdesc[1 - buf]:
                                dd.wait()
                            sdesc[1 - buf] = None
                        gdesc[1 - buf] = pltpu.async_copy(
                            h_hbm.at[edc_v.at[j + 1, 0]], rowsv[1 - buf],
                            semg[1 - buf])
                    gdesc[buf].wait()
                    for k in range(B // LANES):
                        exv[buf][pl.ds(k * LANES, LANES)] = jnp.exp(
                            alpha16(j, k) - G)

                    def prow(r2, cc2, _buf=buf):
                        for u in range(2):
                            exb = plsc.load_gather(
                                exv[_buf],
                                [jnp.full((LANES,), 2 * r2 + u,
                                          dtype=jnp.int32)])
                            for k in range(D // LANES):
                                rowsv[_buf][
                                    2 * r2 + u, pl.ds(k * LANES, LANES)] = (
                                    rowsv[_buf][2 * r2 + u,
                                                pl.ds(k * LANES, LANES)]
                                    * exb)
                        return cc2
                    lax.fori_loop(0, B // 2, prow, 0)
                    sdesc[buf] = (
                        pltpu.async_copy(rowsv[buf],
                                         acc_s.at[edc_v.at[j, 1]],
                                         sems[buf], add=True),
                        pltpu.async_copy(exv[buf],
                                         den_s.at[edc_v.at[j, 1]],
                                         sems[buf], add=True))
                for b in (0, 1):
                    if sdesc[b] is not None:
                        for dd in sdesc[b]:
                            dd.wait()
            lax.fori_loop(0, nch,
                          lambda cc, c: (p2_chunk(cc * ch, ch), c)[1], 0)
            if tail:
                p2_chunk(nch * ch, tail)

            plsc.subcore_barrier()
            _dump_stripe(acc_s, den_s, acc_out, den_out, cid, sid, rz)

    return gat_sc


def _make_sage_sc(n_pad):
    steps = E_RC_PAD // NTILES // B
    rz = n_pad // 16

    @functools.partial(
        pl.kernel,
        out_type=[jax.ShapeDtypeStruct((2, n_pad, D), jnp.float32),
                  jax.ShapeDtypeStruct((2, n_pad), jnp.float32)],
        mesh=_SC_MESH,
        scratch_types=[
            pltpu.VMEM((steps, 3, B), jnp.int32),
            pltpu.VMEM((B,), jnp.float32),
            pltpu.VMEM((B,), jnp.float32),
            pltpu.VMEM((B, D), jnp.float32),
            pltpu.VMEM((B, D), jnp.float32),
            pltpu.VMEM_SHARED((n_pad, D), jnp.float32),
            pltpu.VMEM_SHARED((n_pad,), jnp.float32),
            pltpu.SemaphoreType.DMA,
            pltpu.SemaphoreType.DMA,
            pltpu.SemaphoreType.DMA,
            pltpu.SemaphoreType.DMA,
        ],
        compiler_params=_SC_PARAMS,
    )
    def sage_sc(ed_hbm, x_hbm, acc_out, den_out,
                edc_v, ones_v, zeros_v, rows0_v, rows1_v, acc_s, den_s,
                semg0, semg1, sems0, sems1):
        cid = lax.axis_index("c")
        sid = lax.axis_index("s")
        wid = cid * 16 + sid
        rowsv = (rows0_v, rows1_v)
        semg = (semg0, semg1)
        sems = (sems0, sems1)
        for k in range(B // LANES):
            ones_v[pl.ds(k * LANES, LANES)] = jnp.full((LANES,), 1.0,
                                                       jnp.float32)
        _zero_stripe(rows0_v, zeros_v, acc_s, den_s, sid, rz)
        plsc.subcore_barrier()
        pltpu.sync_copy(ed_hbm.at[wid], edc_v)
        gdesc = [None, None]
        sdesc = [None, None]
        gdesc[0] = pltpu.async_copy(x_hbm.at[edc_v.at[0, 0]],
                                    rows0_v, semg[0])
        for j in range(steps):
            buf = j % 2
            if j + 1 < steps:
                if sdesc[1 - buf] is not None:
                    for dd in sdesc[1 - buf]:
                        dd.wait()
                    sdesc[1 - buf] = None
                gdesc[1 - buf] = pltpu.async_copy(
                    x_hbm.at[edc_v.at[j + 1, 0]], rowsv[1 - buf],
                    semg[1 - buf])
            gdesc[buf].wait()
            sdesc[buf] = (
                pltpu.async_copy(rowsv[buf], acc_s.at[edc_v.at[j, 1]],
                                 sems[buf], add=True),
                pltpu.async_copy(ones_v, den_s.at[edc_v.at[j, 1]],
                                 sems[buf], add=True))
        for b in (0, 1):
            if sdesc[b] is not None:
                for dd in sdesc[b]:
                    dd.wait()
        plsc.subcore_barrier()
        _dump_stripe(acc_s, den_s, acc_out, den_out, cid, sid, rz)

    return sage_sc


_gat_sc_ruru = _make_gat_sc(N_REST, N_REST, N_PAD_RU, E_UR)
_sage_sc_cr = _make_sage_sc(N_PAD_RU)
_sage_sc_rc = _make_sage_sc(N_PAD_C)


def _prep_body(xs_ref, xd_ref, ws_ref, avs_ref, wd_ref, avd_ref,
               h_ref, asrc_ref, adst_ref):
    h = jnp.dot(xs_ref[...], ws_ref[...], preferred_element_type=jnp.float32)
    h_ref[...] = h
    asrc_ref[...] = jnp.dot(h, avs_ref[...], preferred_element_type=jnp.float32)
    wv = jnp.dot(wd_ref[...], avd_ref[...], preferred_element_type=jnp.float32)
    adst_ref[...] = jnp.dot(xd_ref[...], wv, preferred_element_type=jnp.float32)


def _tc_prep(x_src, x_dst, p):
    n_s, n_d = x_src.shape[0], x_dst.shape[0]
    h, a_s, a_d = pl.pallas_call(
        _prep_body,
        out_shape=[jax.ShapeDtypeStruct((n_s, D), jnp.float32),
                   jax.ShapeDtypeStruct((n_s, 1), jnp.float32),
                   jax.ShapeDtypeStruct((n_d, 1), jnp.float32)],
    )(x_src, x_dst, p['W_src'], p['att_src'].reshape(D, 1),
      p['W_dst'], p['att_dst'].reshape(D, 1))
    return h, a_s.reshape(n_s), a_d.reshape(n_d)


def _edge_body(ea_ref, we_ref, ave_ref, out_ref):
    we = jnp.dot(we_ref[...], ave_ref[...],
                 preferred_element_type=jnp.float32)
    wet = jnp.concatenate([we] * 8, axis=0)
    r = lax.broadcasted_iota(jnp.int32, (128, 8), 0)
    cidx = lax.broadcasted_iota(jnp.int32, (128, 8), 1)
    wsel = jnp.where(r // 16 == cidx, wet, 0.0)
    out_ref[...] = jnp.dot(ea_ref[...], wsel,
                           preferred_element_type=jnp.float32)


def _tc_edge(edge_attr, p):
    e = edge_attr.shape[0]
    ea2 = edge_attr.reshape(e // 8, 128)
    out = pl.pallas_call(
        _edge_body,
        out_shape=jax.ShapeDtypeStruct((e // 8, 8), jnp.float32),
    )(ea2, p['W_edge'], p['att_edge'].reshape(D, 1))
    return out.reshape(e)


def _gat_from_acc(gacc_ref, gden_ref, g_ref, n):
    g0 = g_ref[0, 0]
    g1 = g_ref[1, 0]
    gm = jnp.maximum(g0, g1)
    s0 = jnp.exp(g0 - gm)
    s1 = jnp.exp(g1 - gm)
    f = gacc_ref[0, :n, :] * s0 + gacc_ref[1, :n, :] * s1
    den = gden_ref[0, :n, :] * s0 + gden_ref[1, :n, :] * s1
    return f / (den + 1e-16)


def _sage_from_acc(sacc_ref, scnt_ref, x_ref, wl_ref, wr_ref, bs_ref, n):
    ss = sacc_ref[0, :n, :] + sacc_ref[1, :n, :]
    cnt = scnt_ref[0, :n, :] + scnt_ref[1, :n, :]
    mean = ss / jnp.maximum(cnt, 1.0)
    return (jnp.dot(mean, wl_ref[...], preferred_element_type=jnp.float32)
            + jnp.dot(x_ref[...], wr_ref[...],
                      preferred_element_type=jnp.float32)
            + bs_ref[...])


def _bn_relu(y, gamma_ref, beta_ref):
    mu = jnp.mean(y, axis=0, keepdims=True)
    var = jnp.mean((y - mu) * (y - mu), axis=0, keepdims=True)
    out = (y - mu) / jnp.sqrt(var + 1e-5) * gamma_ref[...] + beta_ref[...]
    return jnp.maximum(out, 0.0)


def _comb_rest_body(gacc_ref, gden_ref, g_ref, sacc_ref, scnt_ref, x_ref,
                    bg_ref, wl_ref, wr_ref, bs_ref, gamma_ref, beta_ref,
                    out_ref):
    gat = _gat_from_acc(gacc_ref, gden_ref, g_ref, N_REST) + bg_ref[...]
    sage = _sage_from_acc(sacc_ref, scnt_ref, x_ref, wl_ref, wr_ref, bs_ref,
                          N_REST)
    out_ref[...] = _bn_relu(gat + sage, gamma_ref, beta_ref)


def _comb_user_body(gacc_ref, gden_ref, g_ref, bg_ref, gamma_ref, beta_ref,
                    out_ref):
    gat = _gat_from_acc(gacc_ref, gden_ref, g_ref, N_USER) + bg_ref[...]
    out_ref[...] = _bn_relu(gat, gamma_ref, beta_ref)


def _comb_cat_body(sacc_ref, scnt_ref, x_ref, wl_ref, wr_ref, bs_ref,
                   gamma_ref, beta_ref, out_ref):
    sage = _sage_from_acc(sacc_ref, scnt_ref, x_ref, wl_ref, wr_ref, bs_ref,
                          N_CAT)
    out_ref[...] = _bn_relu(sage, gamma_ref, beta_ref)


_SMEM_SPEC = pl.BlockSpec(memory_space=pltpu.SMEM)


def _col(den):
    return den.reshape(den.shape[0], den.shape[1], 1)


def _comb_rest(gacc, gden, g2, sacc, scnt, x, pg, ps, bn):
    return pl.pallas_call(
        _comb_rest_body,
        out_shape=jax.ShapeDtypeStruct((N_REST, D), jnp.float32),
        in_specs=[pl.BlockSpec(None), pl.BlockSpec(None), _SMEM_SPEC]
        + [pl.BlockSpec(None)] * 9,
    )(gacc, _col(gden), g2, sacc, _col(scnt), x, pg['bias'].reshape(1, D),
      ps['W_l'], ps['W_r'], ps['bias'].reshape(1, D),
      bn['gamma'].reshape(1, D), bn['beta'].reshape(1, D))


def _comb_user(gacc, gden, g2, pg, bn):
    return pl.pallas_call(
        _comb_user_body,
        out_shape=jax.ShapeDtypeStruct((N_USER, D), jnp.float32),
        in_specs=[pl.BlockSpec(None), pl.BlockSpec(None), _SMEM_SPEC]
        + [pl.BlockSpec(None)] * 3,
    )(gacc, _col(gden), g2, pg['bias'].reshape(1, D),
      bn['gamma'].reshape(1, D), bn['beta'].reshape(1, D))


def _comb_cat(sacc, scnt, x, ps, bn):
    return pl.pallas_call(
        _comb_cat_body,
        out_shape=jax.ShapeDtypeStruct((N_CAT, D), jnp.float32),
    )(sacc, _col(scnt), x, ps['W_l'], ps['W_r'], ps['bias'].reshape(1, D),
      bn['gamma'].reshape(1, D), bn['beta'].reshape(1, D))


def _pack_edges(src, dst, ae):
    steps = src.shape[0] // (NTILES * B)
    return jnp.stack(
        [src.astype(jnp.int32).reshape(NTILES, steps, B),
         dst.astype(jnp.int32).reshape(NTILES, steps, B),
         lax.bitcast_convert_type(ae, jnp.int32).reshape(NTILES, steps, B)],
        axis=2)


def _pad_idx(idx, fill):
    pad = E_RC_PAD - E_RC
    return jnp.concatenate([idx.astype(jnp.int32),
                            jnp.full((pad,), fill, dtype=jnp.int32)])


def kernel(x_user, x_restaurant, x_category, edge_index_ur, edge_index_ru,
           edge_index_rc, edge_index_cr, edge_attr_ur, edge_attr_ru, params):
    src_cr = _pad_idx(edge_index_cr[0], 0)
    dst_cr = _pad_idx(edge_index_cr[1], N_REST)
    src_rc = _pad_idx(edge_index_rc[0], 0)
    dst_rc = _pad_idx(edge_index_rc[1], N_CAT)
    z_ae = jnp.zeros((E_RC_PAD,), jnp.float32)
    ed_cr = _pack_edges(src_cr, dst_cr, z_ae)
    ed_rc = _pack_edges(src_rc, dst_rc, z_ae)

    xu, xr, xc = x_user, x_restaurant, x_category
    for lname in ('l1', 'l2'):
        P = params[lname]
        h_u, as_u, ad_r = _tc_prep(xu, xr, P['gat_ur'])
        h_r, as_r, ad_u = _tc_prep(xr, xu, P['gat_ru'])
        ae_ur = _tc_edge(edge_attr_ur, P['gat_ur'])
        ae_ru = _tc_edge(edge_attr_ru, P['gat_ru'])
        ed_ur = _pack_edges(edge_index_ur[0], edge_index_ur[1], ae_ur)
        ed_ru = _pack_edges(edge_index_ru[0], edge_index_ru[1], ae_ru)

        gacc_r, gden_r, g_r = _gat_sc_ruru(ed_ur, as_u, ad_r, h_u)
        gacc_u, gden_u, g_u = _gat_sc_ruru(ed_ru, as_r, ad_u, h_r)
        sacc_r, scnt_r = _sage_sc_cr(ed_cr, xc)
        sacc_c, scnt_c = _sage_sc_rc(ed_rc, xr)

        xr_new = _comb_rest(gacc_r, gden_r, g_r, sacc_r, scnt_r, xr,
                            P['gat_ur'], P['sage_cr'], P['bn']['restaurant'])
        xu_new = _comb_user(gacc_u, gden_u, g_u, P['gat_ru'], P['bn']['user'])
        xc_new = _comb_cat(sacc_c, scnt_c, xc, P['sage_rc'],
                           P['bn']['category'])
        xu, xr, xc = xu_new, xr_new, xc_new

    return (xu, xr, xc)


# speedup vs baseline: 1.1374x; 1.0032x over previous
import functools

import jax
import jax.numpy as jnp
from jax import lax
from jax.experimental import pallas as pl
from jax.experimental.pallas import tpu as pltpu
from jax.experimental.pallas import tpu_sc as plsc

N_USER = 10000
N_REST = 10000
N_CAT = 512
D = 128
D_EDGE = 16
E_UR = 320000
E_RC = 40000

NTILES = 32
LANES = 16
B = 80

N_PAD_RU = 10112
N_PAD_C = 640
E_RC_PAD = 40960


_SC_MESH = plsc.VectorSubcoreMesh(core_axis_name="c", subcore_axis_name="s",
                                  num_cores=2, num_subcores=16)
_SC_PARAMS = pltpu.CompilerParams(use_tc_tiling_on_sc=False,
                                  needs_layout_passes=False)


def _zero_stripe(rows0_v, ex0_v, acc_s, den_s, sid, rz):
    def zrow(r, c):
        for k in range(D // LANES):
            rows0_v[r, pl.ds(k * LANES, LANES)] = jnp.zeros(
                (LANES,), jnp.float32)
        return c
    lax.fori_loop(0, B, zrow, 0)
    for k in range(B // LANES):
        ex0_v[pl.ds(k * LANES, LANES)] = jnp.zeros((LANES,), jnp.float32)
    base = sid * rz
    off = 0
    while off < rz:
        n = min(B, rz - off)
        pltpu.sync_copy(rows0_v.at[pl.ds(0, n)],
                        acc_s.at[pl.ds(base + off, n)])
        pltpu.sync_copy(ex0_v.at[pl.ds(0, n)],
                        den_s.at[pl.ds(base + off, n)])
        off += n


def _dump_stripe(acc_s, den_s, acc_out, den_out, cid, sid, rz):
    pltpu.sync_copy(acc_s.at[pl.ds(sid * rz, rz)],
                    acc_out.at[cid, pl.ds(sid * rz, rz)])
    pltpu.sync_copy(den_s.at[pl.ds(sid * rz, rz)],
                    den_out.at[cid, pl.ds(sid * rz, rz)])


def _make_gat_sc(n_src, n_dst, n_pad, e_total):
    e_t = e_total // NTILES
    steps = e_t // B
    ch = 10 if steps >= 10 else 8
    nch = steps // ch
    tail = steps - nch * ch
    rz = n_pad // 16

    @functools.partial(
        pl.kernel,
        out_type=[jax.ShapeDtypeStruct((2, n_pad, D), jnp.float32),
                  jax.ShapeDtypeStruct((2, n_pad), jnp.float32),
                  jax.ShapeDtypeStruct((2, 16), jnp.float32)],
        mesh=_SC_MESH,
        scratch_types=[
            pltpu.VMEM((n_src,), jnp.float32),
            pltpu.VMEM((n_dst,), jnp.float32),
            pltpu.VMEM((ch, 3, B), jnp.int32),
            pltpu.VMEM((B,), jnp.float32),
            pltpu.VMEM((B,), jnp.float32),
            pltpu.VMEM((B, D), jnp.float32),
            pltpu.VMEM((B, D), jnp.float32),
            pltpu.VMEM((16,), jnp.float32),
            pltpu.VMEM((16, 16), jnp.float32),
            pltpu.VMEM_SHARED((16, 16), jnp.float32),
            pltpu.VMEM_SHARED((n_pad, D), jnp.float32),
            pltpu.VMEM_SHARED((n_pad,), jnp.float32),
            pltpu.SemaphoreType.DMA,
            pltpu.SemaphoreType.DMA,
            pltpu.SemaphoreType.DMA,
            pltpu.SemaphoreType.DMA,
        ],
        compiler_params=_SC_PARAMS,
    )
    def gat_sc(ed_hbm, asrc_hbm, adst_hbm, h_hbm,
               acc_out, den_out, g_out,
               asrc_v, adst_v, edc_v, ex0_v, ex1_v, rows0_v, rows1_v,
               tmp16_v, gmax_l, gmax_s, acc_s, den_s,
               semg0, semg1, sems0, sems1):
        cid = lax.axis_index("c")
        sid = lax.axis_index("s")
        wid = cid * 16 + sid
        exv = (ex0_v, ex1_v)
        rowsv = (rows0_v, rows1_v)
        semg = (semg0, semg1)
        sems = (sems0, sems1)

        if True:
            pltpu.sync_copy(asrc_hbm, asrc_v)
            pltpu.sync_copy(adst_hbm, adst_v)
            _zero_stripe(rows0_v, ex0_v, acc_s, den_s, sid, rz)

            def alpha16(j, k):
                i16 = edc_v[j, 0, pl.ds(k * LANES, LANES)]
                d16 = edc_v[j, 1, pl.ds(k * LANES, LANES)]
                ae = plsc.bitcast(edc_v[j, 2, pl.ds(k * LANES, LANES)],
                                  jnp.float32)
                a = (plsc.load_gather(asrc_v, [i16])
                     + plsc.load_gather(adst_v, [d16]) + ae)
                return jnp.maximum(a, 0.0) + 0.2 * jnp.minimum(a, 0.0)

            def p1_chunk(cbase, mx, nsteps):
                pltpu.sync_copy(ed_hbm.at[wid, pl.ds(cbase, nsteps)],
                                edc_v.at[pl.ds(0, nsteps)])
                for j in range(nsteps):
                    for k in range(B // LANES):
                        mx = jnp.maximum(mx, alpha16(j, k))
                return mx
            mx = lax.fori_loop(
                0, nch, lambda cc, mx: p1_chunk(cc * ch, mx, ch),
                jnp.full((LANES,), -1e30, dtype=jnp.float32))
            if tail:
                mx = p1_chunk(nch * ch, mx, tail)
            tmp16_v[...] = mx
            pltpu.sync_copy(tmp16_v, gmax_s.at[sid])
            plsc.subcore_barrier()
            pltpu.sync_copy(gmax_s, gmax_l)
            gm = gmax_l[0, pl.ds(0, LANES)]
            for r in range(1, 16):
                gm = jnp.maximum(gm, gmax_l[r, pl.ds(0, LANES)])
            G = lax.reduce_max(gm, axes=(0,))

            @pl.when(sid == 0)
            def _():
                tmp16_v[...] = jnp.full((LANES,), G, dtype=jnp.float32)
                pltpu.sync_copy(tmp16_v, g_out.at[cid])

            def p2_chunk(cbase, nsteps):
                pltpu.sync_copy(ed_hbm.at[wid, pl.ds(cbase, nsteps)],
                                edc_v.at[pl.ds(0, nsteps)])
                gdesc = [None, None]
                sdesc = [None, None]
                gdesc[0] = pltpu.async_copy(h_hbm.at[edc_v.at[0, 0]],
                                            rows0_v, semg[0])
                for j in range(nsteps):
                    buf = j % 2
                    if j + 1 < nsteps:
                        if sdesc[1 - buf] is not None:
                            for dd in sdesc[1 - buf]:
                                dd.wait()
                            sdesc[1 - buf] = None
                        gdesc[1 - buf] = pltpu.async_copy(
                            h_hbm.at[edc_v.at[j + 1, 0]], rowsv[1 - buf],
                            semg[1 - buf])
                    gdesc[buf].wait()
                    for k in range(B // LANES):
                        exv[buf][pl.ds(k * LANES, LANES)] = jnp.exp(
                            alpha16(j, k) - G)

                    def prow(r2, cc2, _buf=buf):
                        for u in range(4):
                            exb = plsc.load_gather(
                                exv[_buf],
                                [jnp.full((LANES,), 4 * r2 + u,
                                          dtype=jnp.int32)])
                            for k in range(D // LANES):
                                rowsv[_buf][
                                    4 * r2 + u, pl.ds(k * LANES, LANES)] = (
                                    rowsv[_buf][4 * r2 + u,
                                                pl.ds(k * LANES, LANES)]
                                    * exb)
                        return cc2
                    lax.fori_loop(0, B // 4, prow, 0)
                    sdesc[buf] = (
                        pltpu.async_copy(rowsv[buf],
                                         acc_s.at[edc_v.at[j, 1]],
                                         sems[buf], add=True),
                        pltpu.async_copy(exv[buf],
                                         den_s.at[edc_v.at[j, 1]],
                                         sems[buf], add=True))
                for b in (0, 1):
                    if sdesc[b] is not None:
                        for dd in sdesc[b]:
                            dd.wait()
            lax.fori_loop(0, nch,
                          lambda cc, c: (p2_chunk(cc * ch, ch), c)[1], 0)
            if tail:
                p2_chunk(nch * ch, tail)

            plsc.subcore_barrier()
            _dump_stripe(acc_s, den_s, acc_out, den_out, cid, sid, rz)

    return gat_sc


def _make_sage_sc(n_pad):
    steps = E_RC_PAD // NTILES // B
    rz = n_pad // 16

    @functools.partial(
        pl.kernel,
        out_type=[jax.ShapeDtypeStruct((2, n_pad, D), jnp.float32),
                  jax.ShapeDtypeStruct((2, n_pad), jnp.float32)],
        mesh=_SC_MESH,
        scratch_types=[
            pltpu.VMEM((steps, 3, B), jnp.int32),
            pltpu.VMEM((B,), jnp.float32),
            pltpu.VMEM((B,), jnp.float32),
            pltpu.VMEM((B, D), jnp.float32),
            pltpu.VMEM((B, D), jnp.float32),
            pltpu.VMEM_SHARED((n_pad, D), jnp.float32),
            pltpu.VMEM_SHARED((n_pad,), jnp.float32),
            pltpu.SemaphoreType.DMA,
            pltpu.SemaphoreType.DMA,
            pltpu.SemaphoreType.DMA,
            pltpu.SemaphoreType.DMA,
        ],
        compiler_params=_SC_PARAMS,
    )
    def sage_sc(ed_hbm, x_hbm, acc_out, den_out,
                edc_v, ones_v, zeros_v, rows0_v, rows1_v, acc_s, den_s,
                semg0, semg1, sems0, sems1):
        cid = lax.axis_index("c")
        sid = lax.axis_index("s")
        wid = cid * 16 + sid
        rowsv = (rows0_v, rows1_v)
        semg = (semg0, semg1)
        sems = (sems0, sems1)
        for k in range(B // LANES):
            ones_v[pl.ds(k * LANES, LANES)] = jnp.full((LANES,), 1.0,
                                                       jnp.float32)
        _zero_stripe(rows0_v, zeros_v, acc_s, den_s, sid, rz)
        plsc.subcore_barrier()
        pltpu.sync_copy(ed_hbm.at[wid], edc_v)
        gdesc = [None, None]
        sdesc = [None, None]
        gdesc[0] = pltpu.async_copy(x_hbm.at[edc_v.at[0, 0]],
                                    rows0_v, semg[0])
        for j in range(steps):
            buf = j % 2
            if j + 1 < steps:
                if sdesc[1 - buf] is not None:
                    for dd in sdesc[1 - buf]:
                        dd.wait()
                    sdesc[1 - buf] = None
                gdesc[1 - buf] = pltpu.async_copy(
                    x_hbm.at[edc_v.at[j + 1, 0]], rowsv[1 - buf],
                    semg[1 - buf])
            gdesc[buf].wait()
            sdesc[buf] = (
                pltpu.async_copy(rowsv[buf], acc_s.at[edc_v.at[j, 1]],
                                 sems[buf], add=True),
                pltpu.async_copy(ones_v, den_s.at[edc_v.at[j, 1]],
                                 sems[buf], add=True))
        for b in (0, 1):
            if sdesc[b] is not None:
                for dd in sdesc[b]:
                    dd.wait()
        plsc.subcore_barrier()
        _dump_stripe(acc_s, den_s, acc_out, den_out, cid, sid, rz)

    return sage_sc


_gat_sc_ruru = _make_gat_sc(N_REST, N_REST, N_PAD_RU, E_UR)
_sage_sc_cr = _make_sage_sc(N_PAD_RU)
_sage_sc_rc = _make_sage_sc(N_PAD_C)


def _prep_body(xs_ref, xd_ref, ws_ref, avs_ref, wd_ref, avd_ref,
               h_ref, asrc_ref, adst_ref):
    h = jnp.dot(xs_ref[...], ws_ref[...], preferred_element_type=jnp.float32)
    h_ref[...] = h
    asrc_ref[...] = jnp.dot(h, avs_ref[...], preferred_element_type=jnp.float32)
    wv = jnp.dot(wd_ref[...], avd_ref[...], preferred_element_type=jnp.float32)
    adst_ref[...] = jnp.dot(xd_ref[...], wv, preferred_element_type=jnp.float32)


def _tc_prep(x_src, x_dst, p):
    n_s, n_d = x_src.shape[0], x_dst.shape[0]
    h, a_s, a_d = pl.pallas_call(
        _prep_body,
        out_shape=[jax.ShapeDtypeStruct((n_s, D), jnp.float32),
                   jax.ShapeDtypeStruct((n_s, 1), jnp.float32),
                   jax.ShapeDtypeStruct((n_d, 1), jnp.float32)],
    )(x_src, x_dst, p['W_src'], p['att_src'].reshape(D, 1),
      p['W_dst'], p['att_dst'].reshape(D, 1))
    return h, a_s.reshape(n_s), a_d.reshape(n_d)


def _edge_body(ea_ref, we_ref, ave_ref, out_ref):
    we = jnp.dot(we_ref[...], ave_ref[...],
                 preferred_element_type=jnp.float32)
    wet = jnp.concatenate([we] * 8, axis=0)
    r = lax.broadcasted_iota(jnp.int32, (128, 8), 0)
    cidx = lax.broadcasted_iota(jnp.int32, (128, 8), 1)
    wsel = jnp.where(r // 16 == cidx, wet, 0.0)
    out_ref[...] = jnp.dot(ea_ref[...], wsel,
                           preferred_element_type=jnp.float32)


def _tc_edge(edge_attr, p):
    e = edge_attr.shape[0]
    ea2 = edge_attr.reshape(e // 8, 128)
    out = pl.pallas_call(
        _edge_body,
        out_shape=jax.ShapeDtypeStruct((e // 8, 8), jnp.float32),
    )(ea2, p['W_edge'], p['att_edge'].reshape(D, 1))
    return out.reshape(e)


def _gat_from_acc(gacc_ref, gden_ref, g_ref, n):
    g0 = g_ref[0, 0]
    g1 = g_ref[1, 0]
    gm = jnp.maximum(g0, g1)
    s0 = jnp.exp(g0 - gm)
    s1 = jnp.exp(g1 - gm)
    f = gacc_ref[0, :n, :] * s0 + gacc_ref[1, :n, :] * s1
    den = gden_ref[0, :n, :] * s0 + gden_ref[1, :n, :] * s1
    return f / (den + 1e-16)


def _sage_from_acc(sacc_ref, scnt_ref, x_ref, wl_ref, wr_ref, bs_ref, n):
    ss = sacc_ref[0, :n, :] + sacc_ref[1, :n, :]
    cnt = scnt_ref[0, :n, :] + scnt_ref[1, :n, :]
    mean = ss / jnp.maximum(cnt, 1.0)
    return (jnp.dot(mean, wl_ref[...], preferred_element_type=jnp.float32)
            + jnp.dot(x_ref[...], wr_ref[...],
                      preferred_element_type=jnp.float32)
            + bs_ref[...])


def _bn_relu(y, gamma_ref, beta_ref):
    mu = jnp.mean(y, axis=0, keepdims=True)
    var = jnp.mean((y - mu) * (y - mu), axis=0, keepdims=True)
    out = (y - mu) / jnp.sqrt(var + 1e-5) * gamma_ref[...] + beta_ref[...]
    return jnp.maximum(out, 0.0)


def _comb_rest_body(gacc_ref, gden_ref, g_ref, sacc_ref, scnt_ref, x_ref,
                    bg_ref, wl_ref, wr_ref, bs_ref, gamma_ref, beta_ref,
                    out_ref):
    gat = _gat_from_acc(gacc_ref, gden_ref, g_ref, N_REST) + bg_ref[...]
    sage = _sage_from_acc(sacc_ref, scnt_ref, x_ref, wl_ref, wr_ref, bs_ref,
                          N_REST)
    out_ref[...] = _bn_relu(gat + sage, gamma_ref, beta_ref)


def _comb_user_body(gacc_ref, gden_ref, g_ref, bg_ref, gamma_ref, beta_ref,
                    out_ref):
    gat = _gat_from_acc(gacc_ref, gden_ref, g_ref, N_USER) + bg_ref[...]
    out_ref[...] = _bn_relu(gat, gamma_ref, beta_ref)


def _comb_cat_body(sacc_ref, scnt_ref, x_ref, wl_ref, wr_ref, bs_ref,
                   gamma_ref, beta_ref, out_ref):
    sage = _sage_from_acc(sacc_ref, scnt_ref, x_ref, wl_ref, wr_ref, bs_ref,
                          N_CAT)
    out_ref[...] = _bn_relu(sage, gamma_ref, beta_ref)


_SMEM_SPEC = pl.BlockSpec(memory_space=pltpu.SMEM)


def _col(den):
    return den.reshape(den.shape[0], den.shape[1], 1)


def _comb_rest(gacc, gden, g2, sacc, scnt, x, pg, ps, bn):
    return pl.pallas_call(
        _comb_rest_body,
        out_shape=jax.ShapeDtypeStruct((N_REST, D), jnp.float32),
        in_specs=[pl.BlockSpec(None), pl.BlockSpec(None), _SMEM_SPEC]
        + [pl.BlockSpec(None)] * 9,
    )(gacc, _col(gden), g2, sacc, _col(scnt), x, pg['bias'].reshape(1, D),
      ps['W_l'], ps['W_r'], ps['bias'].reshape(1, D),
      bn['gamma'].reshape(1, D), bn['beta'].reshape(1, D))


def _comb_user(gacc, gden, g2, pg, bn):
    return pl.pallas_call(
        _comb_user_body,
        out_shape=jax.ShapeDtypeStruct((N_USER, D), jnp.float32),
        in_specs=[pl.BlockSpec(None), pl.BlockSpec(None), _SMEM_SPEC]
        + [pl.BlockSpec(None)] * 3,
    )(gacc, _col(gden), g2, pg['bias'].reshape(1, D),
      bn['gamma'].reshape(1, D), bn['beta'].reshape(1, D))


def _comb_cat(sacc, scnt, x, ps, bn):
    return pl.pallas_call(
        _comb_cat_body,
        out_shape=jax.ShapeDtypeStruct((N_CAT, D), jnp.float32),
    )(sacc, _col(scnt), x, ps['W_l'], ps['W_r'], ps['bias'].reshape(1, D),
      bn['gamma'].reshape(1, D), bn['beta'].reshape(1, D))


def _pack_edges(src, dst, ae):
    steps = src.shape[0] // (NTILES * B)
    return jnp.stack(
        [src.astype(jnp.int32).reshape(NTILES, steps, B),
         dst.astype(jnp.int32).reshape(NTILES, steps, B),
         lax.bitcast_convert_type(ae, jnp.int32).reshape(NTILES, steps, B)],
        axis=2)


def _pad_idx(idx, fill):
    pad = E_RC_PAD - E_RC
    return jnp.concatenate([idx.astype(jnp.int32),
                            jnp.full((pad,), fill, dtype=jnp.int32)])


def kernel(x_user, x_restaurant, x_category, edge_index_ur, edge_index_ru,
           edge_index_rc, edge_index_cr, edge_attr_ur, edge_attr_ru, params):
    src_cr = _pad_idx(edge_index_cr[0], 0)
    dst_cr = _pad_idx(edge_index_cr[1], N_REST)
    src_rc = _pad_idx(edge_index_rc[0], 0)
    dst_rc = _pad_idx(edge_index_rc[1], N_CAT)
    z_ae = jnp.zeros((E_RC_PAD,), jnp.float32)
    ed_cr = _pack_edges(src_cr, dst_cr, z_ae)
    ed_rc = _pack_edges(src_rc, dst_rc, z_ae)

    xu, xr, xc = x_user, x_restaurant, x_category
    for lname in ('l1', 'l2'):
        P = params[lname]
        h_u, as_u, ad_r = _tc_prep(xu, xr, P['gat_ur'])
        h_r, as_r, ad_u = _tc_prep(xr, xu, P['gat_ru'])
        ae_ur = _tc_edge(edge_attr_ur, P['gat_ur'])
        ae_ru = _tc_edge(edge_attr_ru, P['gat_ru'])
        ed_ur = _pack_edges(edge_index_ur[0], edge_index_ur[1], ae_ur)
        ed_ru = _pack_edges(edge_index_ru[0], edge_index_ru[1], ae_ru)

        gacc_r, gden_r, g_r = _gat_sc_ruru(ed_ur, as_u, ad_r, h_u)
        gacc_u, gden_u, g_u = _gat_sc_ruru(ed_ru, as_r, ad_u, h_r)
        sacc_r, scnt_r = _sage_sc_cr(ed_cr, xc)
        sacc_c, scnt_c = _sage_sc_rc(ed_rc, xr)

        xr_new = _comb_rest(gacc_r, gden_r, g_r, sacc_r, scnt_r, xr,
                            P['gat_ur'], P['sage_cr'], P['bn']['restaurant'])
        xu_new = _comb_user(gacc_u, gden_u, g_u, P['gat_ru'], P['bn']['user'])
        xc_new = _comb_cat(sacc_c, scnt_c, xc, P['sage_rc'],
                           P['bn']['category'])
        xu, xr, xc = xu_new, xr_new, xc_new

    return (xu, xr, xc)
